# baseline (speedup 1.0000x reference)
import functools

import numpy as np
import jax
import jax.numpy as jnp
from jax.experimental import pallas as pl
from jax.experimental.pallas import tpu as pltpu

_LANE = 128
_BB = 32


def _round_up(x, m):
    return int(pl.cdiv(x, m) * m)


@functools.lru_cache(maxsize=None)
def _triu_tables(N, P_pad):
    assert N == _LANE
    R = P_pad // _LANE
    idx = np.zeros((N, N), np.int32)
    m1 = np.zeros((N, N), np.float32)
    m2 = np.zeros((N, N), np.float32)
    s1 = np.zeros((R, N), np.float32)
    s2 = np.zeros((R, N), np.float32)
    start = 0
    for i in range(N - 1):
        ln = N - 1 - i
        l0 = start % _LANE
        r0 = start // _LANE
        shift = (l0 - (i + 1)) % N
        idx[i, :] = (np.arange(N) - shift) % N
        len1 = min(ln, _LANE - l0)
        m1[i, l0:l0 + len1] = 1.0
        s1[r0, i] = 1.0
        if ln > len1:
            m2[i, 0:ln - len1] = 1.0
            s2[r0 + 1, i] = 1.0
        start += ln
    s12 = np.concatenate([s1, s2], axis=1)
    eye = np.eye(N, dtype=np.float32)
    return idx, m1, m2, s12, eye


def _make_body(BB, N, F, E):
    hp = jax.lax.Precision.DEFAULT

    def _body(thr_ref, nf_ref, sc_ref, idx_ref, m1_ref, m2_ref, s12_ref,
              eye_ref, w0_ref, b0_ref, w1_ref, b1_ref, w2_ref, b2_ref,
              sc_out_ref, dist_ref):
        thr = thr_ref[0, 0]
        sc_out_ref[...] = sc_ref[...] - thr

        x = nf_ref[...].reshape(BB * N, F)
        h = jnp.dot(x, w0_ref[...], precision=hp,
                    preferred_element_type=jnp.float32) + b0_ref[...]
        h = jnp.maximum(h, 0.0)
        h = jnp.dot(h, w1_ref[...], precision=hp,
                    preferred_element_type=jnp.float32) + b1_ref[...]
        h = jnp.maximum(h, 0.0)
        h = jnp.dot(h, w2_ref[...], precision=hp,
                    preferred_element_type=jnp.float32) + b2_ref[...]
        h3 = h.reshape(BB, N, E)

        idx = idx_ref[...]
        m1 = m1_ref[...]
        m2 = m2_ref[...]
        eye = eye_ref[...]

        s12 = s12_ref[...]
        for e in range(BB):
            X = h3[e]
            G = jax.lax.dot_general(
                X, X, (((1,), (1,)), ((), ())), precision=hp,
                preferred_element_type=jnp.float32)
            Gd = G * eye
            sq_col = jnp.sum(Gd, axis=1, keepdims=True)
            sq_row = jnp.sum(Gd, axis=0, keepdims=True)
            D = (sq_col + sq_row) - 2.0 * G
            A = jnp.take_along_axis(D, idx, axis=1)
            A12 = jnp.concatenate([A * m1, A * m2], axis=0)
            dist_ref[e] = jnp.dot(s12, A12,
                                  preferred_element_type=jnp.float32)

    return _body


def kernel(sc, node_features, sc_threshold, w0, b0, w1, b1, w2, b2,
           triu_rows, triu_cols, pair_diff_t):
    B1, B2, N, F = node_features.shape
    B = B1 * B2
    P = sc.shape[-1]
    E = w2.shape[1]
    W = w0.shape[1]
    P_pad = _round_up(P, _LANE)
    R = P_pad // _LANE
    BB = _BB
    assert B % BB == 0 and N == _LANE

    idx_np, m1_np, m2_np, s12_np, eye_np = _triu_tables(N, P_pad)
    idx = jnp.asarray(idx_np)
    m1 = jnp.asarray(m1_np)
    m2 = jnp.asarray(m2_np)
    s12 = jnp.asarray(s12_np)
    eye = jnp.asarray(eye_np)

    nf = node_features.reshape(B, N, F)
    scf = sc.reshape(B, P)
    thr = sc_threshold.reshape(1, 1)
    b0r = b0.reshape(1, -1)
    b1r = b1.reshape(1, -1)
    b2r = b2.reshape(1, -1)

    mlp_flops = 2 * BB * N * (F * W + W * W + W * E)
    gram_flops = BB * 2 * N * N * E
    route_flops = 2 * R * 2 * N * BB * N
    cost = pl.CostEstimate(
        flops=int((B // BB) * (mlp_flops + gram_flops + route_flops)),
        transcendentals=0,
        bytes_accessed=int(4 * (nf.size + 2 * scf.size + B * R * _LANE)),
    )

    full = lambda shape: pl.BlockSpec(shape, lambda i: tuple(0 for _ in shape))
    sc_out, dist = pl.pallas_call(
        _make_body(BB, N, F, E),
        out_shape=(jax.ShapeDtypeStruct((B, P), sc.dtype),
                   jax.ShapeDtypeStruct((B, R, _LANE), node_features.dtype)),
        grid=(B // BB,),
        in_specs=[
            pl.BlockSpec((1, 1), lambda i: (0, 0),
                         memory_space=pltpu.MemorySpace.SMEM),
            pl.BlockSpec((BB, N, F), lambda i: (i, 0, 0)),
            pl.BlockSpec((BB, P), lambda i: (i, 0)),
            full((N, N)),
            full((N, N)),
            full((N, N)),
            full((R, 2 * N)),
            full((N, N)),
            full((F, W)), full((1, W)),
            full((W, W)), full((1, W)),
            full((W, E)), full((1, E)),
        ],
        out_specs=(
            pl.BlockSpec((BB, P), lambda i: (i, 0)),
            pl.BlockSpec((BB, R, _LANE), lambda i: (i, 0, 0)),
        ),
        compiler_params=pltpu.CompilerParams(
            dimension_semantics=("parallel",),
            vmem_limit_bytes=64 * 1024 * 1024,
        ),
        cost_estimate=cost,
    )(thr, nf, scf, idx, m1, m2, s12, eye, w0, b0r, w1, b1r, w2, b2r)

    sc_out = sc_out.reshape(B1, B2, P)
    dists = dist.reshape(B, R * _LANE)[:, :P].reshape(B1, B2, P)
    return sc_out, dists


# speedup vs baseline: 1.6441x; 1.6441x over previous
import functools

import numpy as np
import jax
import jax.numpy as jnp
from jax.experimental import pallas as pl
from jax.experimental.pallas import tpu as pltpu

_LANE = 128
_BB = 32


def _round_up(x, m):
    return int(pl.cdiv(x, m) * m)


@functools.lru_cache(maxsize=None)
def _triu_tables(N, P_pad):
    assert N == _LANE
    R = P_pad // _LANE
    idx = np.zeros((N, N), np.int32)
    m1 = np.zeros((N, N), np.float32)
    m2 = np.zeros((N, N), np.float32)
    s1 = np.zeros((R, N), np.float32)
    s2 = np.zeros((R, N), np.float32)
    start = 0
    for i in range(N - 1):
        ln = N - 1 - i
        l0 = start % _LANE
        r0 = start // _LANE
        shift = (l0 - (i + 1)) % N
        idx[i, :] = (np.arange(N) - shift) % N
        len1 = min(ln, _LANE - l0)
        m1[i, l0:l0 + len1] = 1.0
        s1[r0, i] = 1.0
        if ln > len1:
            m2[i, 0:ln - len1] = 1.0
            s2[r0 + 1, i] = 1.0
        start += ln
    s12 = np.concatenate([s1, s2], axis=1)
    eye = np.eye(N, dtype=np.float32)
    return idx, m1, m2, s12, eye


def _make_body(BB, N, F, E):
    hp = jax.lax.Precision.DEFAULT

    def _body(thr_ref, nf_ref, sc_ref, idx_ref, m1_ref, m2_ref, s12_ref,
              eye_ref, w0_ref, b0_ref, w1_ref, b1_ref, w2_ref, b2_ref,
              sc_out_ref, dist_ref):
        thr = thr_ref[0, 0]
        sc_out_ref[...] = sc_ref[...] - thr

        x = nf_ref[...].reshape(BB * N, F)
        h = jnp.dot(x, w0_ref[...], precision=hp,
                    preferred_element_type=jnp.float32) + b0_ref[...]
        h = jnp.maximum(h, 0.0)
        h = jnp.dot(h, w1_ref[...], precision=hp,
                    preferred_element_type=jnp.float32) + b1_ref[...]
        h = jnp.maximum(h, 0.0)
        h = jnp.dot(h, w2_ref[...], precision=hp,
                    preferred_element_type=jnp.float32) + b2_ref[...]
        h3 = h.reshape(BB, N, E)

        idx = idx_ref[...]
        m1 = m1_ref[...]
        m2 = m2_ref[...]
        eye = eye_ref[...]

        s12 = s12_ref[...]
        CH = 8
        for c in range(0, BB, CH):
            parts1 = []
            parts2 = []
            for e in range(c, c + CH):
                X = h3[e]
                G = jax.lax.dot_general(
                    X, X, (((1,), (1,)), ((), ())), precision=hp,
                    preferred_element_type=jnp.float32)
                Gd = G * eye
                sq_col = jnp.sum(Gd, axis=1, keepdims=True)
                sq_row = jnp.sum(Gd, axis=0, keepdims=True)
                D = (sq_col + sq_row) - 2.0 * G
                A = jnp.take_along_axis(D, idx, axis=1)
                parts1.append(A * m1)
                parts2.append(A * m2)
            A12 = jnp.concatenate(
                [jnp.concatenate(parts1, axis=1),
                 jnp.concatenate(parts2, axis=1)], axis=0)
            out_wide = jnp.dot(s12, A12,
                               preferred_element_type=jnp.float32)
            for k in range(CH):
                dist_ref[c + k] = out_wide[:, k * _LANE:(k + 1) * _LANE]

    return _body


def kernel(sc, node_features, sc_threshold, w0, b0, w1, b1, w2, b2,
           triu_rows, triu_cols, pair_diff_t):
    B1, B2, N, F = node_features.shape
    B = B1 * B2
    P = sc.shape[-1]
    E = w2.shape[1]
    W = w0.shape[1]
    P_pad = _round_up(P, _LANE)
    R = P_pad // _LANE
    BB = _BB
    assert B % BB == 0 and N == _LANE

    idx_np, m1_np, m2_np, s12_np, eye_np = _triu_tables(N, P_pad)
    idx = jnp.asarray(idx_np)
    m1 = jnp.asarray(m1_np)
    m2 = jnp.asarray(m2_np)
    s12 = jnp.asarray(s12_np)
    eye = jnp.asarray(eye_np)

    nf = node_features.reshape(B, N, F)
    scf = sc.reshape(B, P)
    thr = sc_threshold.reshape(1, 1)
    b0r = b0.reshape(1, -1)
    b1r = b1.reshape(1, -1)
    b2r = b2.reshape(1, -1)

    mlp_flops = 2 * BB * N * (F * W + W * W + W * E)
    gram_flops = BB * 2 * N * N * E
    route_flops = 2 * R * 2 * N * BB * N
    cost = pl.CostEstimate(
        flops=int((B // BB) * (mlp_flops + gram_flops + route_flops)),
        transcendentals=0,
        bytes_accessed=int(4 * (nf.size + 2 * scf.size + B * R * _LANE)),
    )

    full = lambda shape: pl.BlockSpec(shape, lambda i: tuple(0 for _ in shape))
    sc_out, dist = pl.pallas_call(
        _make_body(BB, N, F, E),
        out_shape=(jax.ShapeDtypeStruct((B, P), sc.dtype),
                   jax.ShapeDtypeStruct((B, R, _LANE), node_features.dtype)),
        grid=(B // BB,),
        in_specs=[
            pl.BlockSpec((1, 1), lambda i: (0, 0),
                         memory_space=pltpu.MemorySpace.SMEM),
            pl.BlockSpec((BB, N, F), lambda i: (i, 0, 0)),
            pl.BlockSpec((BB, P), lambda i: (i, 0)),
            full((N, N)),
            full((N, N)),
            full((N, N)),
            full((R, 2 * N)),
            full((N, N)),
            full((F, W)), full((1, W)),
            full((W, W)), full((1, W)),
            full((W, E)), full((1, E)),
        ],
        out_specs=(
            pl.BlockSpec((BB, P), lambda i: (i, 0)),
            pl.BlockSpec((BB, R, _LANE), lambda i: (i, 0, 0)),
        ),
        compiler_params=pltpu.CompilerParams(
            dimension_semantics=("parallel",),
            vmem_limit_bytes=64 * 1024 * 1024,
        ),
        cost_estimate=cost,
    )(thr, nf, scf, idx, m1, m2, s12, eye, w0, b0r, w1, b1r, w2, b2r)

    sc_out = sc_out.reshape(B1, B2, P)
    dists = dist.reshape(B, R * _LANE)[:, :P].reshape(B1, B2, P)
    return sc_out, dists


# speedup vs baseline: 1.7935x; 1.0908x over previous
import functools

import numpy as np
import jax
import jax.numpy as jnp
from jax.experimental import pallas as pl
from jax.experimental.pallas import tpu as pltpu

_LANE = 128
_BB = 32


def _round_up(x, m):
    return int(pl.cdiv(x, m) * m)


@functools.lru_cache(maxsize=None)
def _triu_tables(N, P_pad):
    assert N == _LANE
    R = P_pad // _LANE
    idx = np.zeros((N, N), np.int32)
    m1 = np.zeros((N, N), np.float32)
    m2 = np.zeros((N, N), np.float32)
    s1 = np.zeros((R, N), np.float32)
    s2 = np.zeros((R, N), np.float32)
    start = 0
    for i in range(N - 1):
        ln = N - 1 - i
        l0 = start % _LANE
        r0 = start // _LANE
        shift = (l0 - (i + 1)) % N
        idx[i, :] = (np.arange(N) - shift) % N
        len1 = min(ln, _LANE - l0)
        m1[i, l0:l0 + len1] = 1.0
        s1[r0, i] = 1.0
        if ln > len1:
            m2[i, 0:ln - len1] = 1.0
            s2[r0 + 1, i] = 1.0
        start += ln
    s12 = np.concatenate([s1, s2], axis=1)
    eye = np.eye(N, dtype=np.float32)
    return idx, m1, m2, s12, eye


def _make_body(BB, N, F, E):
    hp = jax.lax.Precision.DEFAULT

    def _body(thr_ref, nf_ref, sc_ref, idx_ref, m1_ref, m2_ref, s12_ref,
              eye_ref, w0_ref, b0_ref, w1_ref, b1_ref, w2_ref, b2_ref,
              sc_out_ref, dist_ref):
        thr = thr_ref[0, 0]
        sc_out_ref[...] = sc_ref[...] - thr

        x = nf_ref[...].reshape(BB * N, F)
        h = jnp.dot(x, w0_ref[...], precision=hp,
                    preferred_element_type=jnp.float32) + b0_ref[...]
        h = jnp.maximum(h, 0.0)
        h = jnp.dot(h, w1_ref[...], precision=hp,
                    preferred_element_type=jnp.float32) + b1_ref[...]
        h = jnp.maximum(h, 0.0)
        h = jnp.dot(h, w2_ref[...], precision=hp,
                    preferred_element_type=jnp.float32) + b2_ref[...]
        h3 = h.reshape(BB, N, E)

        idx = idx_ref[...]
        m1 = m1_ref[...]
        m2 = m2_ref[...]
        eye = eye_ref[...]

        s12 = s12_ref[...]
        jones = jnp.ones((N, N), jnp.float32)
        CH = 8
        for c in range(0, BB, CH):
            parts1 = []
            parts2 = []
            for e in range(c, c + CH):
                X = h3[e]
                G = jax.lax.dot_general(
                    X, X, (((1,), (1,)), ((), ())), precision=hp,
                    preferred_element_type=jnp.float32)
                Gd = G * eye
                sq_row_b = jnp.dot(jones, Gd, precision=hp,
                                   preferred_element_type=jnp.float32)
                sq_col_b = jnp.dot(Gd, jones, precision=hp,
                                   preferred_element_type=jnp.float32)
                D = (sq_col_b + sq_row_b) - 2.0 * G
                A = jnp.take_along_axis(D, idx, axis=1)
                parts1.append(A * m1)
                parts2.append(A * m2)
            A12 = jnp.concatenate(
                [jnp.concatenate(parts1, axis=1),
                 jnp.concatenate(parts2, axis=1)], axis=0)
            out_wide = jnp.dot(s12, A12,
                               preferred_element_type=jnp.float32)
            for k in range(CH):
                dist_ref[c + k] = out_wide[:, k * _LANE:(k + 1) * _LANE]

    return _body


def kernel(sc, node_features, sc_threshold, w0, b0, w1, b1, w2, b2,
           triu_rows, triu_cols, pair_diff_t):
    B1, B2, N, F = node_features.shape
    B = B1 * B2
    P = sc.shape[-1]
    E = w2.shape[1]
    W = w0.shape[1]
    P_pad = _round_up(P, _LANE)
    R = P_pad // _LANE
    BB = _BB
    assert B % BB == 0 and N == _LANE

    idx_np, m1_np, m2_np, s12_np, eye_np = _triu_tables(N, P_pad)
    idx = jnp.asarray(idx_np)
    m1 = jnp.asarray(m1_np)
    m2 = jnp.asarray(m2_np)
    s12 = jnp.asarray(s12_np)
    eye = jnp.asarray(eye_np)

    nf = node_features.reshape(B, N, F)
    scf = sc.reshape(B, P)
    thr = sc_threshold.reshape(1, 1)
    b0r = b0.reshape(1, -1)
    b1r = b1.reshape(1, -1)
    b2r = b2.reshape(1, -1)

    mlp_flops = 2 * BB * N * (F * W + W * W + W * E)
    gram_flops = BB * 2 * N * N * E
    route_flops = 2 * R * 2 * N * BB * N
    cost = pl.CostEstimate(
        flops=int((B // BB) * (mlp_flops + gram_flops + route_flops)),
        transcendentals=0,
        bytes_accessed=int(4 * (nf.size + 2 * scf.size + B * R * _LANE)),
    )

    full = lambda shape: pl.BlockSpec(shape, lambda i: tuple(0 for _ in shape))
    sc_out, dist = pl.pallas_call(
        _make_body(BB, N, F, E),
        out_shape=(jax.ShapeDtypeStruct((B, P), sc.dtype),
                   jax.ShapeDtypeStruct((B, R, _LANE), node_features.dtype)),
        grid=(B // BB,),
        in_specs=[
            pl.BlockSpec((1, 1), lambda i: (0, 0),
                         memory_space=pltpu.MemorySpace.SMEM),
            pl.BlockSpec((BB, N, F), lambda i: (i, 0, 0)),
            pl.BlockSpec((BB, P), lambda i: (i, 0)),
            full((N, N)),
            full((N, N)),
            full((N, N)),
            full((R, 2 * N)),
            full((N, N)),
            full((F, W)), full((1, W)),
            full((W, W)), full((1, W)),
            full((W, E)), full((1, E)),
        ],
        out_specs=(
            pl.BlockSpec((BB, P), lambda i: (i, 0)),
            pl.BlockSpec((BB, R, _LANE), lambda i: (i, 0, 0)),
        ),
        compiler_params=pltpu.CompilerParams(
            dimension_semantics=("parallel",),
            vmem_limit_bytes=64 * 1024 * 1024,
        ),
        cost_estimate=cost,
    )(thr, nf, scf, idx, m1, m2, s12, eye, w0, b0r, w1, b1r, w2, b2r)

    sc_out = sc_out.reshape(B1, B2, P)
    dists = dist.reshape(B, R * _LANE)[:, :P].reshape(B1, B2, P)
    return sc_out, dists


# speedup vs baseline: 1.8020x; 1.0048x over previous
import functools

import numpy as np
import jax
import jax.numpy as jnp
from jax.experimental import pallas as pl
from jax.experimental.pallas import tpu as pltpu

_LANE = 128
_BB = 32


def _round_up(x, m):
    return int(pl.cdiv(x, m) * m)


@functools.lru_cache(maxsize=None)
def _triu_tables(N, P_pad):
    assert N == _LANE
    R = P_pad // _LANE
    idx = np.zeros((N, N), np.int32)
    m1 = np.zeros((N, N), np.float32)
    m2 = np.zeros((N, N), np.float32)
    s1 = np.zeros((R, N), np.float32)
    s2 = np.zeros((R, N), np.float32)
    start = 0
    for i in range(N - 1):
        ln = N - 1 - i
        l0 = start % _LANE
        r0 = start // _LANE
        shift = (l0 - (i + 1)) % N
        idx[i, :] = (np.arange(N) - shift) % N
        len1 = min(ln, _LANE - l0)
        m1[i, l0:l0 + len1] = 1.0
        s1[r0, i] = 1.0
        if ln > len1:
            m2[i, 0:ln - len1] = 1.0
            s2[r0 + 1, i] = 1.0
        start += ln
    s12 = np.concatenate([s1, s2], axis=1)
    eye = np.eye(N, dtype=np.float32)
    return idx, m1, m2, s12, eye


def _make_body(BB, N, F, E):
    hp = jax.lax.Precision.DEFAULT

    def _body(thr_ref, nf_ref, sc_ref, idx_ref, m1_ref, m2_ref, s12_ref,
              eye_ref, w0_ref, b0_ref, w1_ref, b1_ref, w2_ref, b2_ref,
              sc_out_ref, dist_ref):
        thr = thr_ref[0, 0]
        sc_out_ref[...] = sc_ref[...] - thr

        bf16 = jnp.bfloat16
        x = nf_ref[...].reshape(BB * N, F).astype(bf16)
        h = jnp.dot(x, w0_ref[...].astype(bf16),
                    preferred_element_type=jnp.float32) + b0_ref[...]
        h = jnp.maximum(h, 0.0).astype(bf16)
        h = jnp.dot(h, w1_ref[...].astype(bf16),
                    preferred_element_type=jnp.float32) + b1_ref[...]
        h = jnp.maximum(h, 0.0).astype(bf16)
        h = jnp.dot(h, w2_ref[...].astype(bf16),
                    preferred_element_type=jnp.float32) + b2_ref[...]
        h3 = h.reshape(BB, N, E)

        idx = idx_ref[...]
        m1 = m1_ref[...]
        m2 = m2_ref[...]
        eye = eye_ref[...]

        s12 = s12_ref[...]
        jones = jnp.ones((N, N), jnp.float32)
        CH = 8
        for c in range(0, BB, CH):
            parts1 = []
            parts2 = []
            for e in range(c, c + CH):
                X = h3[e]
                G = jax.lax.dot_general(
                    X, X, (((1,), (1,)), ((), ())), precision=hp,
                    preferred_element_type=jnp.float32)
                Gd = G * eye
                sq_row_b = jnp.dot(jones, Gd, precision=hp,
                                   preferred_element_type=jnp.float32)
                sq_col_b = jnp.dot(Gd, jones, precision=hp,
                                   preferred_element_type=jnp.float32)
                D = (sq_col_b + sq_row_b) - 2.0 * G
                A = jnp.take_along_axis(D, idx, axis=1)
                parts1.append(A * m1)
                parts2.append(A * m2)
            A12 = jnp.concatenate(
                [jnp.concatenate(parts1, axis=1),
                 jnp.concatenate(parts2, axis=1)], axis=0)
            out_wide = jnp.dot(s12, A12,
                               preferred_element_type=jnp.float32)
            for k in range(CH):
                dist_ref[c + k] = out_wide[:, k * _LANE:(k + 1) * _LANE]

    return _body


def kernel(sc, node_features, sc_threshold, w0, b0, w1, b1, w2, b2,
           triu_rows, triu_cols, pair_diff_t):
    B1, B2, N, F = node_features.shape
    B = B1 * B2
    P = sc.shape[-1]
    E = w2.shape[1]
    W = w0.shape[1]
    P_pad = _round_up(P, _LANE)
    R = P_pad // _LANE
    BB = _BB
    assert B % BB == 0 and N == _LANE

    idx_np, m1_np, m2_np, s12_np, eye_np = _triu_tables(N, P_pad)
    idx = jnp.asarray(idx_np)
    m1 = jnp.asarray(m1_np)
    m2 = jnp.asarray(m2_np)
    s12 = jnp.asarray(s12_np)
    eye = jnp.asarray(eye_np)

    nf = node_features.reshape(B, N, F)
    scf = sc.reshape(B, P)
    thr = sc_threshold.reshape(1, 1)
    b0r = b0.reshape(1, -1)
    b1r = b1.reshape(1, -1)
    b2r = b2.reshape(1, -1)

    mlp_flops = 2 * BB * N * (F * W + W * W + W * E)
    gram_flops = BB * 2 * N * N * E
    route_flops = 2 * R * 2 * N * BB * N
    cost = pl.CostEstimate(
        flops=int((B // BB) * (mlp_flops + gram_flops + route_flops)),
        transcendentals=0,
        bytes_accessed=int(4 * (nf.size + 2 * scf.size + B * R * _LANE)),
    )

    full = lambda shape: pl.BlockSpec(shape, lambda i: tuple(0 for _ in shape))
    sc_out, dist = pl.pallas_call(
        _make_body(BB, N, F, E),
        out_shape=(jax.ShapeDtypeStruct((B, P), sc.dtype),
                   jax.ShapeDtypeStruct((B, R, _LANE), node_features.dtype)),
        grid=(B // BB,),
        in_specs=[
            pl.BlockSpec((1, 1), lambda i: (0, 0),
                         memory_space=pltpu.MemorySpace.SMEM),
            pl.BlockSpec((BB, N, F), lambda i: (i, 0, 0)),
            pl.BlockSpec((BB, P), lambda i: (i, 0)),
            full((N, N)),
            full((N, N)),
            full((N, N)),
            full((R, 2 * N)),
            full((N, N)),
            full((F, W)), full((1, W)),
            full((W, W)), full((1, W)),
            full((W, E)), full((1, E)),
        ],
        out_specs=(
            pl.BlockSpec((BB, P), lambda i: (i, 0)),
            pl.BlockSpec((BB, R, _LANE), lambda i: (i, 0, 0)),
        ),
        compiler_params=pltpu.CompilerParams(
            dimension_semantics=("parallel",),
            vmem_limit_bytes=64 * 1024 * 1024,
        ),
        cost_estimate=cost,
    )(thr, nf, scf, idx, m1, m2, s12, eye, w0, b0r, w1, b1r, w2, b2r)

    sc_out = sc_out.reshape(B1, B2, P)
    dists = dist.reshape(B, R * _LANE)[:, :P].reshape(B1, B2, P)
    return sc_out, dists


# speedup vs baseline: 2.0545x; 1.1401x over previous
import functools

import numpy as np
import jax
import jax.numpy as jnp
from jax.experimental import pallas as pl
from jax.experimental.pallas import tpu as pltpu

_LANE = 128
_BB = 32


def _round_up(x, m):
    return int(pl.cdiv(x, m) * m)


@functools.lru_cache(maxsize=None)
def _triu_tables(N, P_pad):
    assert N == _LANE
    R = P_pad // _LANE
    idx = np.zeros((N, N), np.int32)
    m1 = np.zeros((N, N), np.float32)
    m2 = np.zeros((N, N), np.float32)
    s1 = np.zeros((R, N), np.float32)
    s2 = np.zeros((R, N), np.float32)
    start = 0
    for i in range(N - 1):
        ln = N - 1 - i
        l0 = start % _LANE
        r0 = start // _LANE
        shift = (l0 - (i + 1)) % N
        idx[i, :] = (np.arange(N) - shift) % N
        len1 = min(ln, _LANE - l0)
        m1[i, l0:l0 + len1] = 1.0
        s1[r0, i] = 1.0
        if ln > len1:
            m2[i, 0:ln - len1] = 1.0
            s2[r0 + 1, i] = 1.0
        start += ln
    s12 = np.concatenate([s1, s2], axis=1)
    return idx, m1, m2, s12


def _make_body(BB, N, F, E):
    def _body(thr_ref, nf_ref, sc_ref, idx_ref, m1_ref, m2_ref, s12_ref,
              w0_ref, b0_ref, w1_ref, b1_ref, w2_ref, b2_ref,
              sc_out_ref, dist_ref):
        thr = thr_ref[0, 0]
        sc_out_ref[...] = sc_ref[...] - thr

        bf16 = jnp.bfloat16
        x = nf_ref[...].reshape(BB * N, F).astype(bf16)
        h = jnp.dot(x, w0_ref[...].astype(bf16),
                    preferred_element_type=jnp.float32) + b0_ref[...]
        h = jnp.maximum(h, 0.0).astype(bf16)
        h = jnp.dot(h, w1_ref[...].astype(bf16),
                    preferred_element_type=jnp.float32) + b1_ref[...]
        h = jnp.maximum(h, 0.0).astype(bf16)
        h = jnp.dot(h, w2_ref[...].astype(bf16),
                    preferred_element_type=jnp.float32) + b2_ref[...]
        h3b = h.astype(bf16).reshape(BB, N, E)

        idx = idx_ref[...]
        m1 = m1_ref[...]
        m2 = m2_ref[...]

        s12 = s12_ref[...]
        jones = jnp.ones((N, E), bf16)
        cdims = (((1,), (1,)), ((), ()))
        CH = 8
        for c in range(0, BB, CH):
            parts1 = []
            parts2 = []
            for e in range(c, c + CH):
                Xb = h3b[e]
                G = jax.lax.dot_general(
                    Xb, Xb, cdims,
                    preferred_element_type=jnp.float32)
                HHb = Xb * Xb
                sq_col_b = jax.lax.dot_general(
                    HHb, jones, cdims,
                    preferred_element_type=jnp.float32)
                sq_row_b = jax.lax.dot_general(
                    jones, HHb, cdims,
                    preferred_element_type=jnp.float32)
                D = (sq_col_b + sq_row_b) - 2.0 * G
                A = jnp.take_along_axis(D, idx, axis=1)
                parts1.append(A * m1)
                parts2.append(A * m2)
            A12 = jnp.concatenate(
                [jnp.concatenate(parts1, axis=1),
                 jnp.concatenate(parts2, axis=1)], axis=0)
            out_wide = jnp.dot(s12, A12,
                               preferred_element_type=jnp.float32)
            for k in range(CH):
                dist_ref[c + k] = out_wide[:, k * _LANE:(k + 1) * _LANE]

    return _body


def kernel(sc, node_features, sc_threshold, w0, b0, w1, b1, w2, b2,
           triu_rows, triu_cols, pair_diff_t):
    B1, B2, N, F = node_features.shape
    B = B1 * B2
    P = sc.shape[-1]
    E = w2.shape[1]
    W = w0.shape[1]
    P_pad = _round_up(P, _LANE)
    R = P_pad // _LANE
    BB = _BB
    assert B % BB == 0 and N == _LANE

    idx_np, m1_np, m2_np, s12_np = _triu_tables(N, P_pad)
    idx = jnp.asarray(idx_np)
    m1 = jnp.asarray(m1_np)
    m2 = jnp.asarray(m2_np)
    s12 = jnp.asarray(s12_np)

    nf = node_features.reshape(B, N, F)
    scf = sc.reshape(B, P)
    thr = sc_threshold.reshape(1, 1)
    b0r = b0.reshape(1, -1)
    b1r = b1.reshape(1, -1)
    b2r = b2.reshape(1, -1)

    mlp_flops = 2 * BB * N * (F * W + W * W + W * E)
    gram_flops = BB * 2 * N * N * E
    route_flops = 2 * R * 2 * N * BB * N
    cost = pl.CostEstimate(
        flops=int((B // BB) * (mlp_flops + gram_flops + route_flops)),
        transcendentals=0,
        bytes_accessed=int(4 * (nf.size + 2 * scf.size + B * R * _LANE)),
    )

    full = lambda shape: pl.BlockSpec(shape, lambda i: tuple(0 for _ in shape))
    sc_out, dist = pl.pallas_call(
        _make_body(BB, N, F, E),
        out_shape=(jax.ShapeDtypeStruct((B, P), sc.dtype),
                   jax.ShapeDtypeStruct((B, R, _LANE), node_features.dtype)),
        grid=(B // BB,),
        in_specs=[
            pl.BlockSpec((1, 1), lambda i: (0, 0),
                         memory_space=pltpu.MemorySpace.SMEM),
            pl.BlockSpec((BB, N, F), lambda i: (i, 0, 0)),
            pl.BlockSpec((BB, P), lambda i: (i, 0)),
            full((N, N)),
            full((N, N)),
            full((N, N)),
            full((R, 2 * N)),
            full((F, W)), full((1, W)),
            full((W, W)), full((1, W)),
            full((W, E)), full((1, E)),
        ],
        out_specs=(
            pl.BlockSpec((BB, P), lambda i: (i, 0)),
            pl.BlockSpec((BB, R, _LANE), lambda i: (i, 0, 0)),
        ),
        compiler_params=pltpu.CompilerParams(
            dimension_semantics=("parallel",),
            vmem_limit_bytes=64 * 1024 * 1024,
        ),
        cost_estimate=cost,
    )(thr, nf, scf, idx, m1, m2, s12, w0, b0r, w1, b1r, w2, b2r)

    sc_out = sc_out.reshape(B1, B2, P)
    dists = dist.reshape(B, R * _LANE)[:, :P].reshape(B1, B2, P)
    return sc_out, dists


# speedup vs baseline: 2.3265x; 1.1324x over previous
import functools

import numpy as np
import jax
import jax.numpy as jnp
from jax.experimental import pallas as pl
from jax.experimental.pallas import tpu as pltpu

_LANE = 128
_BB = 32


def _round_up(x, m):
    return int(pl.cdiv(x, m) * m)


@functools.lru_cache(maxsize=None)
def _triu_tables(N, P_pad):
    assert N == _LANE
    R = P_pad // _LANE
    idx = np.zeros((N, N), np.int32)
    m1 = np.zeros((N, N), np.float32)
    m2 = np.zeros((N, N), np.float32)
    s1 = np.zeros((R, N), np.float32)
    s2 = np.zeros((R, N), np.float32)
    start = 0
    for i in range(N - 1):
        ln = N - 1 - i
        l0 = start % _LANE
        r0 = start // _LANE
        shift = (l0 - (i + 1)) % N
        idx[i, :] = (np.arange(N) - shift) % N
        len1 = min(ln, _LANE - l0)
        m1[i, l0:l0 + len1] = 1.0
        s1[r0, i] = 1.0
        if ln > len1:
            m2[i, 0:ln - len1] = 1.0
            s2[r0 + 1, i] = 1.0
        start += ln
    s12 = np.concatenate([s1, s2], axis=1)
    return idx, m1, m2, s12


def _make_body(BB, N, F, E):
    def _body(thr_ref, nf_ref, sc_ref, idx_ref, m1_ref, m2_ref, s12_ref,
              w0_ref, b0_ref, w1_ref, b1_ref, w2_ref, b2_ref,
              sc_out_ref, dist_ref):
        thr = thr_ref[0, 0]
        sc_out_ref[...] = sc_ref[...] - thr

        bf16 = jnp.bfloat16
        x = nf_ref[...].reshape(BB * N, F).astype(bf16)
        h = jnp.dot(x, w0_ref[...].astype(bf16),
                    preferred_element_type=jnp.float32) + b0_ref[...]
        h = jnp.maximum(h, 0.0).astype(bf16)
        h = jnp.dot(h, w1_ref[...].astype(bf16),
                    preferred_element_type=jnp.float32) + b1_ref[...]
        h = jnp.maximum(h, 0.0).astype(bf16)
        h = jnp.dot(h, w2_ref[...].astype(bf16),
                    preferred_element_type=jnp.float32) + b2_ref[...]
        h3b = h.astype(bf16).reshape(BB, N, E)

        idx = idx_ref[...]
        m1 = m1_ref[...]
        m2 = m2_ref[...]

        s12 = s12_ref[...]
        jones = jnp.ones((N, E), bf16)
        cdims = (((1,), (1,)), ((), ()))
        CH = 8
        for c in range(0, BB, CH):
            parts1 = []
            parts2 = []
            for e in range(c, c + CH):
                Xb = h3b[e]
                HHb = Xb * Xb
                lhs = jnp.concatenate([HHb, jones, Xb], axis=1)
                rhs = jnp.concatenate([jones, HHb, -2.0 * Xb], axis=1)
                D = jax.lax.dot_general(
                    lhs, rhs, cdims,
                    preferred_element_type=jnp.float32)
                A = jnp.take_along_axis(D, idx, axis=1)
                Ab = A.astype(bf16)
                parts1.append(Ab * m1)
                parts2.append(Ab * m2)
            A12 = jnp.concatenate(
                [jnp.concatenate(parts1, axis=1),
                 jnp.concatenate(parts2, axis=1)], axis=0)
            out_wide = jnp.dot(s12, A12,
                               preferred_element_type=jnp.float32)
            for k in range(CH):
                dist_ref[c + k] = out_wide[:, k * _LANE:(k + 1) * _LANE]

    return _body


def kernel(sc, node_features, sc_threshold, w0, b0, w1, b1, w2, b2,
           triu_rows, triu_cols, pair_diff_t):
    B1, B2, N, F = node_features.shape
    B = B1 * B2
    P = sc.shape[-1]
    E = w2.shape[1]
    W = w0.shape[1]
    P_pad = _round_up(P, _LANE)
    R = P_pad // _LANE
    BB = _BB
    assert B % BB == 0 and N == _LANE

    idx_np, m1_np, m2_np, s12_np = _triu_tables(N, P_pad)
    idx = jnp.asarray(idx_np)
    m1 = jnp.asarray(m1_np, jnp.bfloat16)
    m2 = jnp.asarray(m2_np, jnp.bfloat16)
    s12 = jnp.asarray(s12_np, jnp.bfloat16)

    nf = node_features.reshape(B, N, F)
    scf = sc.reshape(B, P)
    thr = sc_threshold.reshape(1, 1)
    b0r = b0.reshape(1, -1)
    b1r = b1.reshape(1, -1)
    b2r = b2.reshape(1, -1)

    mlp_flops = 2 * BB * N * (F * W + W * W + W * E)
    gram_flops = BB * 2 * N * N * E
    route_flops = 2 * R * 2 * N * BB * N
    cost = pl.CostEstimate(
        flops=int((B // BB) * (mlp_flops + gram_flops + route_flops)),
        transcendentals=0,
        bytes_accessed=int(4 * (nf.size + 2 * scf.size + B * R * _LANE)),
    )

    full = lambda shape: pl.BlockSpec(shape, lambda i: tuple(0 for _ in shape))
    sc_out, dist = pl.pallas_call(
        _make_body(BB, N, F, E),
        out_shape=(jax.ShapeDtypeStruct((B, P), sc.dtype),
                   jax.ShapeDtypeStruct((B, R, _LANE), node_features.dtype)),
        grid=(B // BB,),
        in_specs=[
            pl.BlockSpec((1, 1), lambda i: (0, 0),
                         memory_space=pltpu.MemorySpace.SMEM),
            pl.BlockSpec((BB, N, F), lambda i: (i, 0, 0)),
            pl.BlockSpec((BB, P), lambda i: (i, 0)),
            full((N, N)),
            full((N, N)),
            full((N, N)),
            full((R, 2 * N)),
            full((F, W)), full((1, W)),
            full((W, W)), full((1, W)),
            full((W, E)), full((1, E)),
        ],
        out_specs=(
            pl.BlockSpec((BB, P), lambda i: (i, 0)),
            pl.BlockSpec((BB, R, _LANE), lambda i: (i, 0, 0)),
        ),
        compiler_params=pltpu.CompilerParams(
            dimension_semantics=("parallel",),
            vmem_limit_bytes=64 * 1024 * 1024,
        ),
        cost_estimate=cost,
    )(thr, nf, scf, idx, m1, m2, s12, w0, b0r, w1, b1r, w2, b2r)

    sc_out = sc_out.reshape(B1, B2, P)
    dists = dist.reshape(B, R * _LANE)[:, :P].reshape(B1, B2, P)
    return sc_out, dists


# speedup vs baseline: 2.3269x; 1.0002x over previous
import functools

import numpy as np
import jax
import jax.numpy as jnp
from jax.experimental import pallas as pl
from jax.experimental.pallas import tpu as pltpu

_LANE = 128
_BB = 32


def _round_up(x, m):
    return int(pl.cdiv(x, m) * m)


@functools.lru_cache(maxsize=None)
def _triu_tables(N, P_pad):
    assert N == _LANE
    R = P_pad // _LANE
    idx = np.zeros((N, N), np.int32)
    m1 = np.zeros((N, N), np.float32)
    m2 = np.zeros((N, N), np.float32)
    s1 = np.zeros((R, N), np.float32)
    s2 = np.zeros((R, N), np.float32)
    start = 0
    for i in range(N - 1):
        ln = N - 1 - i
        l0 = start % _LANE
        r0 = start // _LANE
        shift = (l0 - (i + 1)) % N
        idx[i, :] = (np.arange(N) - shift) % N
        len1 = min(ln, _LANE - l0)
        m1[i, l0:l0 + len1] = 1.0
        s1[r0, i] = 1.0
        if ln > len1:
            m2[i, 0:ln - len1] = 1.0
            s2[r0 + 1, i] = 1.0
        start += ln
    s12 = np.concatenate([s1, s2], axis=1)
    return idx, m1, m2, s12


def _make_body(BB, N, F, E):
    def _body(thr_ref, nf_ref, sc_ref, idx_ref, m1_ref, m2_ref, s12_ref,
              w0_ref, b0_ref, w1_ref, b1_ref, w2_ref, b2_ref,
              sc_out_ref, dist_ref):
        thr = thr_ref[0, 0]
        sc_out_ref[...] = sc_ref[...] - thr

        bf16 = jnp.bfloat16
        x = nf_ref[...].reshape(BB * N, F).astype(bf16)
        h = jnp.dot(x, w0_ref[...].astype(bf16),
                    preferred_element_type=jnp.float32)
        h = jnp.maximum(h.astype(bf16) + b0_ref[...].astype(bf16), 0.0)
        h = jnp.dot(h, w1_ref[...].astype(bf16),
                    preferred_element_type=jnp.float32)
        h = jnp.maximum(h.astype(bf16) + b1_ref[...].astype(bf16), 0.0)
        h = jnp.dot(h, w2_ref[...].astype(bf16),
                    preferred_element_type=jnp.float32)
        h3b = (h.astype(bf16) + b2_ref[...].astype(bf16)).reshape(BB, N, E)

        idx = idx_ref[...]
        m1 = m1_ref[...]
        m2 = m2_ref[...]

        s12 = s12_ref[...]
        jones = jnp.ones((N, E), bf16)
        cdims = (((1,), (1,)), ((), ()))
        CH = 8
        for c in range(0, BB, CH):
            parts1 = []
            parts2 = []
            for e in range(c, c + CH):
                Xb = h3b[e]
                HHb = Xb * Xb
                lhs = jnp.concatenate([HHb, jones, Xb], axis=1)
                rhs = jnp.concatenate([jones, HHb, -2.0 * Xb], axis=1)
                D = jax.lax.dot_general(
                    lhs, rhs, cdims,
                    preferred_element_type=jnp.float32)
                A = jnp.take_along_axis(D, idx, axis=1)
                Ab = A.astype(bf16)
                parts1.append(Ab * m1)
                parts2.append(Ab * m2)
            A12 = jnp.concatenate(
                [jnp.concatenate(parts1, axis=1),
                 jnp.concatenate(parts2, axis=1)], axis=0)
            out_wide = jnp.dot(s12, A12,
                               preferred_element_type=jnp.float32)
            for k in range(CH):
                dist_ref[c + k] = out_wide[:, k * _LANE:(k + 1) * _LANE]

    return _body


def kernel(sc, node_features, sc_threshold, w0, b0, w1, b1, w2, b2,
           triu_rows, triu_cols, pair_diff_t):
    B1, B2, N, F = node_features.shape
    B = B1 * B2
    P = sc.shape[-1]
    E = w2.shape[1]
    W = w0.shape[1]
    P_pad = _round_up(P, _LANE)
    R = P_pad // _LANE
    BB = _BB
    assert B % BB == 0 and N == _LANE

    idx_np, m1_np, m2_np, s12_np = _triu_tables(N, P_pad)
    idx = jnp.asarray(idx_np)
    m1 = jnp.asarray(m1_np, jnp.bfloat16)
    m2 = jnp.asarray(m2_np, jnp.bfloat16)
    s12 = jnp.asarray(s12_np, jnp.bfloat16)

    nf = node_features.reshape(B, N, F)
    scf = sc.reshape(B, P)
    thr = sc_threshold.reshape(1, 1)
    b0r = b0.reshape(1, -1)
    b1r = b1.reshape(1, -1)
    b2r = b2.reshape(1, -1)

    mlp_flops = 2 * BB * N * (F * W + W * W + W * E)
    gram_flops = BB * 2 * N * N * E
    route_flops = 2 * R * 2 * N * BB * N
    cost = pl.CostEstimate(
        flops=int((B // BB) * (mlp_flops + gram_flops + route_flops)),
        transcendentals=0,
        bytes_accessed=int(4 * (nf.size + 2 * scf.size + B * R * _LANE)),
    )

    full = lambda shape: pl.BlockSpec(shape, lambda i: tuple(0 for _ in shape))
    sc_out, dist = pl.pallas_call(
        _make_body(BB, N, F, E),
        out_shape=(jax.ShapeDtypeStruct((B, P), sc.dtype),
                   jax.ShapeDtypeStruct((B, R, _LANE), node_features.dtype)),
        grid=(B // BB,),
        in_specs=[
            pl.BlockSpec((1, 1), lambda i: (0, 0),
                         memory_space=pltpu.MemorySpace.SMEM),
            pl.BlockSpec((BB, N, F), lambda i: (i, 0, 0)),
            pl.BlockSpec((BB, P), lambda i: (i, 0)),
            full((N, N)),
            full((N, N)),
            full((N, N)),
            full((R, 2 * N)),
            full((F, W)), full((1, W)),
            full((W, W)), full((1, W)),
            full((W, E)), full((1, E)),
        ],
        out_specs=(
            pl.BlockSpec((BB, P), lambda i: (i, 0)),
            pl.BlockSpec((BB, R, _LANE), lambda i: (i, 0, 0)),
        ),
        compiler_params=pltpu.CompilerParams(
            dimension_semantics=("parallel",),
            vmem_limit_bytes=64 * 1024 * 1024,
        ),
        cost_estimate=cost,
    )(thr, nf, scf, idx, m1, m2, s12, w0, b0r, w1, b1r, w2, b2r)

    sc_out = sc_out.reshape(B1, B2, P)
    dists = dist.reshape(B, R * _LANE)[:, :P].reshape(B1, B2, P)
    return sc_out, dists


# speedup vs baseline: 2.3600x; 1.0142x over previous
import functools

import numpy as np
import jax
import jax.numpy as jnp
from jax.experimental import pallas as pl
from jax.experimental.pallas import tpu as pltpu

_LANE = 128
_BB = 64


def _round_up(x, m):
    return int(pl.cdiv(x, m) * m)


@functools.lru_cache(maxsize=None)
def _triu_tables(N, P_pad):
    assert N == _LANE
    R = P_pad // _LANE
    idx = np.zeros((N, N), np.int32)
    m1 = np.zeros((N, N), np.float32)
    m2 = np.zeros((N, N), np.float32)
    s1 = np.zeros((R, N), np.float32)
    s2 = np.zeros((R, N), np.float32)
    start = 0
    for i in range(N - 1):
        ln = N - 1 - i
        l0 = start % _LANE
        r0 = start // _LANE
        shift = (l0 - (i + 1)) % N
        idx[i, :] = (np.arange(N) - shift) % N
        len1 = min(ln, _LANE - l0)
        m1[i, l0:l0 + len1] = 1.0
        s1[r0, i] = 1.0
        if ln > len1:
            m2[i, 0:ln - len1] = 1.0
            s2[r0 + 1, i] = 1.0
        start += ln
    s12 = np.concatenate([s1, s2], axis=1)
    return idx, m1, m2, s12


def _make_body(BB, N, F, E):
    def _body(thr_ref, nf_ref, sc_ref, idx_ref, m1_ref, m2_ref, s12_ref,
              w0_ref, b0_ref, w1_ref, b1_ref, w2_ref, b2_ref,
              sc_out_ref, dist_ref):
        thr = thr_ref[0, 0]
        sc_out_ref[...] = sc_ref[...] - thr

        bf16 = jnp.bfloat16
        x = nf_ref[...].reshape(BB * N, F).astype(bf16)
        h = jnp.dot(x, w0_ref[...].astype(bf16),
                    preferred_element_type=jnp.float32)
        h = jnp.maximum(h.astype(bf16) + b0_ref[...].astype(bf16), 0.0)
        h = jnp.dot(h, w1_ref[...].astype(bf16),
                    preferred_element_type=jnp.float32)
        h = jnp.maximum(h.astype(bf16) + b1_ref[...].astype(bf16), 0.0)
        h = jnp.dot(h, w2_ref[...].astype(bf16),
                    preferred_element_type=jnp.float32)
        h3b = (h.astype(bf16) + b2_ref[...].astype(bf16)).reshape(BB, N, E)

        idx = idx_ref[...]
        m1 = m1_ref[...]
        m2 = m2_ref[...]

        s12 = s12_ref[...]
        jones = jnp.ones((N, E), bf16)
        cdims = (((1,), (1,)), ((), ()))
        CH = 8
        for c in range(0, BB, CH):
            parts1 = []
            parts2 = []
            for e in range(c, c + CH):
                Xb = h3b[e]
                HHb = Xb * Xb
                lhs = jnp.concatenate([HHb, jones, Xb], axis=1)
                rhs = jnp.concatenate([jones, HHb, -2.0 * Xb], axis=1)
                D = jax.lax.dot_general(
                    lhs, rhs, cdims,
                    preferred_element_type=jnp.float32)
                A = jnp.take_along_axis(D, idx, axis=1)
                Ab = A.astype(bf16)
                parts1.append(Ab * m1)
                parts2.append(Ab * m2)
            A12 = jnp.concatenate(
                [jnp.concatenate(parts1, axis=1),
                 jnp.concatenate(parts2, axis=1)], axis=0)
            out_wide = jnp.dot(s12, A12,
                               preferred_element_type=jnp.float32)
            for k in range(CH):
                dist_ref[c + k] = out_wide[:, k * _LANE:(k + 1) * _LANE]

    return _body


def kernel(sc, node_features, sc_threshold, w0, b0, w1, b1, w2, b2,
           triu_rows, triu_cols, pair_diff_t):
    B1, B2, N, F = node_features.shape
    B = B1 * B2
    P = sc.shape[-1]
    E = w2.shape[1]
    W = w0.shape[1]
    P_pad = _round_up(P, _LANE)
    R = P_pad // _LANE
    BB = _BB
    assert B % BB == 0 and N == _LANE

    idx_np, m1_np, m2_np, s12_np = _triu_tables(N, P_pad)
    idx = jnp.asarray(idx_np)
    m1 = jnp.asarray(m1_np, jnp.bfloat16)
    m2 = jnp.asarray(m2_np, jnp.bfloat16)
    s12 = jnp.asarray(s12_np, jnp.bfloat16)

    nf = node_features.reshape(B, N, F)
    scf = sc.reshape(B, P)
    thr = sc_threshold.reshape(1, 1)
    b0r = b0.reshape(1, -1)
    b1r = b1.reshape(1, -1)
    b2r = b2.reshape(1, -1)

    mlp_flops = 2 * BB * N * (F * W + W * W + W * E)
    gram_flops = BB * 2 * N * N * E
    route_flops = 2 * R * 2 * N * BB * N
    cost = pl.CostEstimate(
        flops=int((B // BB) * (mlp_flops + gram_flops + route_flops)),
        transcendentals=0,
        bytes_accessed=int(4 * (nf.size + 2 * scf.size + B * R * _LANE)),
    )

    full = lambda shape: pl.BlockSpec(shape, lambda i: tuple(0 for _ in shape))
    sc_out, dist = pl.pallas_call(
        _make_body(BB, N, F, E),
        out_shape=(jax.ShapeDtypeStruct((B, P), sc.dtype),
                   jax.ShapeDtypeStruct((B, R, _LANE), node_features.dtype)),
        grid=(B // BB,),
        in_specs=[
            pl.BlockSpec((1, 1), lambda i: (0, 0),
                         memory_space=pltpu.MemorySpace.SMEM),
            pl.BlockSpec((BB, N, F), lambda i: (i, 0, 0)),
            pl.BlockSpec((BB, P), lambda i: (i, 0)),
            full((N, N)),
            full((N, N)),
            full((N, N)),
            full((R, 2 * N)),
            full((F, W)), full((1, W)),
            full((W, W)), full((1, W)),
            full((W, E)), full((1, E)),
        ],
        out_specs=(
            pl.BlockSpec((BB, P), lambda i: (i, 0)),
            pl.BlockSpec((BB, R, _LANE), lambda i: (i, 0, 0)),
        ),
        compiler_params=pltpu.CompilerParams(
            dimension_semantics=("parallel",),
            vmem_limit_bytes=64 * 1024 * 1024,
        ),
        cost_estimate=cost,
    )(thr, nf, scf, idx, m1, m2, s12, w0, b0r, w1, b1r, w2, b2r)

    sc_out = sc_out.reshape(B1, B2, P)
    dists = dist.reshape(B, R * _LANE)[:, :P].reshape(B1, B2, P)
    return sc_out, dists


# speedup vs baseline: 2.5729x; 1.0902x over previous
import functools

import numpy as np
import jax
import jax.numpy as jnp
from jax.experimental import pallas as pl
from jax.experimental.pallas import tpu as pltpu

_LANE = 128
_BB = 64


def _round_up(x, m):
    return int(pl.cdiv(x, m) * m)


@functools.lru_cache(maxsize=None)
def _triu_tables(N, P_pad):
    assert N == _LANE
    R = P_pad // _LANE
    idx = np.zeros((N, N), np.int32)
    m1 = np.zeros((N, N), np.float32)
    m2 = np.zeros((N, N), np.float32)
    s1 = np.zeros((R, N), np.float32)
    s2 = np.zeros((R, N), np.float32)
    start = 0
    for i in range(N - 1):
        ln = N - 1 - i
        l0 = start % _LANE
        r0 = start // _LANE
        shift = (l0 - (i + 1)) % N
        idx[i, :] = (np.arange(N) - shift) % N
        len1 = min(ln, _LANE - l0)
        m1[i, l0:l0 + len1] = 1.0
        s1[r0, i] = 1.0
        if ln > len1:
            m2[i, 0:ln - len1] = 1.0
            s2[r0 + 1, i] = 1.0
        start += ln
    s12 = np.concatenate([s1, s2], axis=1)
    return idx, m1, m2, s12


def _make_body(BB, N, F, E):
    def _body(thr_ref, nf_ref, sc_ref, idx_ref, m1_ref, m2_ref, s12_ref,
              w0_ref, b0_ref, w1_ref, b1_ref, w2_ref, b2_ref,
              sc_out_ref, dist_ref):
        thr = thr_ref[0, 0]
        sc_out_ref[...] = sc_ref[...] - thr

        bf16 = jnp.bfloat16
        x = nf_ref[...].reshape(BB * N, F).astype(bf16)
        h = jnp.dot(x, w0_ref[...].astype(bf16),
                    preferred_element_type=jnp.float32)
        h = jnp.maximum(h.astype(bf16) + b0_ref[...].astype(bf16), 0.0)
        h = jnp.dot(h, w1_ref[...].astype(bf16),
                    preferred_element_type=jnp.float32)
        h = jnp.maximum(h.astype(bf16) + b1_ref[...].astype(bf16), 0.0)
        h = jnp.dot(h, w2_ref[...].astype(bf16),
                    preferred_element_type=jnp.float32)
        h3b = (h.astype(bf16) + b2_ref[...].astype(bf16)).reshape(BB, N, E)

        idx = idx_ref[...]
        m1 = m1_ref[...]
        m2 = m2_ref[...]

        s12 = s12_ref[...]
        jones = jnp.ones((N, E), bf16)
        cdims = (((1,), (1,)), ((), ()))
        CH = 16
        for c in range(0, BB, CH):
            parts1 = []
            parts2 = []
            for e in range(c, c + CH):
                Xb = h3b[e]
                HHb = Xb * Xb
                lhs = jnp.concatenate([HHb, jones, Xb], axis=1)
                rhs = jnp.concatenate([jones, HHb, -2.0 * Xb], axis=1)
                D = jax.lax.dot_general(
                    lhs, rhs, cdims,
                    preferred_element_type=jnp.float32)
                A = jnp.take_along_axis(D, idx, axis=1)
                Ab = A.astype(bf16)
                parts1.append(Ab * m1)
                parts2.append(Ab * m2)
            A12 = jnp.concatenate(
                [jnp.concatenate(parts1, axis=1),
                 jnp.concatenate(parts2, axis=1)], axis=0)
            out_wide = jnp.dot(s12, A12,
                               preferred_element_type=jnp.float32)
            for k in range(CH):
                dist_ref[c + k] = out_wide[:, k * _LANE:(k + 1) * _LANE]

    return _body


def kernel(sc, node_features, sc_threshold, w0, b0, w1, b1, w2, b2,
           triu_rows, triu_cols, pair_diff_t):
    B1, B2, N, F = node_features.shape
    B = B1 * B2
    P = sc.shape[-1]
    E = w2.shape[1]
    W = w0.shape[1]
    P_pad = _round_up(P, _LANE)
    R = P_pad // _LANE
    BB = _BB
    assert B % BB == 0 and N == _LANE

    idx_np, m1_np, m2_np, s12_np = _triu_tables(N, P_pad)
    idx = jnp.asarray(idx_np)
    m1 = jnp.asarray(m1_np, jnp.bfloat16)
    m2 = jnp.asarray(m2_np, jnp.bfloat16)
    s12 = jnp.asarray(s12_np, jnp.bfloat16)

    nf = node_features.reshape(B, N, F)
    scf = sc.reshape(B, P)
    thr = sc_threshold.reshape(1, 1)
    b0r = b0.reshape(1, -1)
    b1r = b1.reshape(1, -1)
    b2r = b2.reshape(1, -1)

    mlp_flops = 2 * BB * N * (F * W + W * W + W * E)
    gram_flops = BB * 2 * N * N * E
    route_flops = 2 * R * 2 * N * BB * N
    cost = pl.CostEstimate(
        flops=int((B // BB) * (mlp_flops + gram_flops + route_flops)),
        transcendentals=0,
        bytes_accessed=int(4 * (nf.size + 2 * scf.size + B * R * _LANE)),
    )

    full = lambda shape: pl.BlockSpec(shape, lambda i: tuple(0 for _ in shape))
    sc_out, dist = pl.pallas_call(
        _make_body(BB, N, F, E),
        out_shape=(jax.ShapeDtypeStruct((B, P), sc.dtype),
                   jax.ShapeDtypeStruct((B, R, _LANE), node_features.dtype)),
        grid=(B // BB,),
        in_specs=[
            pl.BlockSpec((1, 1), lambda i: (0, 0),
                         memory_space=pltpu.MemorySpace.SMEM),
            pl.BlockSpec((BB, N, F), lambda i: (i, 0, 0)),
            pl.BlockSpec((BB, P), lambda i: (i, 0)),
            full((N, N)),
            full((N, N)),
            full((N, N)),
            full((R, 2 * N)),
            full((F, W)), full((1, W)),
            full((W, W)), full((1, W)),
            full((W, E)), full((1, E)),
        ],
        out_specs=(
            pl.BlockSpec((BB, P), lambda i: (i, 0)),
            pl.BlockSpec((BB, R, _LANE), lambda i: (i, 0, 0)),
        ),
        compiler_params=pltpu.CompilerParams(
            dimension_semantics=("parallel",),
            vmem_limit_bytes=64 * 1024 * 1024,
        ),
        cost_estimate=cost,
    )(thr, nf, scf, idx, m1, m2, s12, w0, b0r, w1, b1r, w2, b2r)

    sc_out = sc_out.reshape(B1, B2, P)
    dists = dist.reshape(B, R * _LANE)[:, :P].reshape(B1, B2, P)
    return sc_out, dists


# speedup vs baseline: 2.6885x; 1.0449x over previous
import functools

import numpy as np
import jax
import jax.numpy as jnp
from jax.experimental import pallas as pl
from jax.experimental.pallas import tpu as pltpu

_LANE = 128
_BB = 64


def _round_up(x, m):
    return int(pl.cdiv(x, m) * m)


@functools.lru_cache(maxsize=None)
def _triu_tables(N, P_pad):
    assert N == _LANE
    R = P_pad // _LANE
    idx = np.zeros((N, N), np.int32)
    m1 = np.zeros((N, N), np.float32)
    m2 = np.zeros((N, N), np.float32)
    s1 = np.zeros((R, N), np.float32)
    s2 = np.zeros((R, N), np.float32)
    start = 0
    for i in range(N - 1):
        ln = N - 1 - i
        l0 = start % _LANE
        r0 = start // _LANE
        shift = (l0 - (i + 1)) % N
        idx[i, :] = (np.arange(N) - shift) % N
        len1 = min(ln, _LANE - l0)
        m1[i, l0:l0 + len1] = 1.0
        s1[r0, i] = 1.0
        if ln > len1:
            m2[i, 0:ln - len1] = 1.0
            s2[r0 + 1, i] = 1.0
        start += ln
    s12 = np.concatenate([s1, s2], axis=1)
    return idx, m1, m2, s12


def _make_body(BB, N, F, E):
    def _body(thr_ref, nf_ref, sc_ref, idx_ref, m1_ref, m2_ref, s12_ref,
              w0_ref, b0_ref, w1_ref, b1_ref, w2_ref, b2_ref,
              sc_out_ref, dist_ref):
        thr = thr_ref[0, 0]
        sc_out_ref[...] = sc_ref[...] - thr

        bf16 = jnp.bfloat16
        x = nf_ref[...].reshape(BB * N, F).astype(bf16)
        h = jnp.dot(x, w0_ref[...].astype(bf16),
                    preferred_element_type=jnp.float32)
        h = jnp.maximum(h.astype(bf16) + b0_ref[...].astype(bf16), 0.0)
        h = jnp.dot(h, w1_ref[...].astype(bf16),
                    preferred_element_type=jnp.float32)
        h = jnp.maximum(h.astype(bf16) + b1_ref[...].astype(bf16), 0.0)
        h = jnp.dot(h, w2_ref[...].astype(bf16),
                    preferred_element_type=jnp.float32)
        h3b = (h.astype(bf16) + b2_ref[...].astype(bf16)).reshape(BB, N, E)

        idx = idx_ref[...]
        m1 = m1_ref[...]
        m2 = m2_ref[...]

        s12 = s12_ref[...]
        jones = jnp.ones((N, E), bf16)
        cdims = (((1,), (1,)), ((), ()))
        CH = 32
        for c in range(0, BB, CH):
            parts1 = []
            parts2 = []
            for e in range(c, c + CH):
                Xb = h3b[e]
                HHb = Xb * Xb
                lhs = jnp.concatenate([HHb, jones, Xb], axis=1)
                rhs = jnp.concatenate([jones, HHb, -2.0 * Xb], axis=1)
                D = jax.lax.dot_general(
                    lhs, rhs, cdims,
                    preferred_element_type=jnp.float32)
                A = jnp.take_along_axis(D, idx, axis=1)
                Ab = A.astype(bf16)
                parts1.append(Ab * m1)
                parts2.append(Ab * m2)
            A12 = jnp.concatenate(
                [jnp.concatenate(parts1, axis=1),
                 jnp.concatenate(parts2, axis=1)], axis=0)
            out_wide = jnp.dot(s12, A12,
                               preferred_element_type=jnp.float32)
            for k in range(CH):
                dist_ref[c + k] = out_wide[:, k * _LANE:(k + 1) * _LANE]

    return _body


def kernel(sc, node_features, sc_threshold, w0, b0, w1, b1, w2, b2,
           triu_rows, triu_cols, pair_diff_t):
    B1, B2, N, F = node_features.shape
    B = B1 * B2
    P = sc.shape[-1]
    E = w2.shape[1]
    W = w0.shape[1]
    P_pad = _round_up(P, _LANE)
    R = P_pad // _LANE
    BB = _BB
    assert B % BB == 0 and N == _LANE

    idx_np, m1_np, m2_np, s12_np = _triu_tables(N, P_pad)
    idx = jnp.asarray(idx_np)
    m1 = jnp.asarray(m1_np, jnp.bfloat16)
    m2 = jnp.asarray(m2_np, jnp.bfloat16)
    s12 = jnp.asarray(s12_np, jnp.bfloat16)

    nf = node_features.reshape(B, N, F)
    scf = sc.reshape(B, P)
    thr = sc_threshold.reshape(1, 1)
    b0r = b0.reshape(1, -1)
    b1r = b1.reshape(1, -1)
    b2r = b2.reshape(1, -1)

    mlp_flops = 2 * BB * N * (F * W + W * W + W * E)
    gram_flops = BB * 2 * N * N * E
    route_flops = 2 * R * 2 * N * BB * N
    cost = pl.CostEstimate(
        flops=int((B // BB) * (mlp_flops + gram_flops + route_flops)),
        transcendentals=0,
        bytes_accessed=int(4 * (nf.size + 2 * scf.size + B * R * _LANE)),
    )

    full = lambda shape: pl.BlockSpec(shape, lambda i: tuple(0 for _ in shape))
    sc_out, dist = pl.pallas_call(
        _make_body(BB, N, F, E),
        out_shape=(jax.ShapeDtypeStruct((B, P), sc.dtype),
                   jax.ShapeDtypeStruct((B, R, _LANE), node_features.dtype)),
        grid=(B // BB,),
        in_specs=[
            pl.BlockSpec((1, 1), lambda i: (0, 0),
                         memory_space=pltpu.MemorySpace.SMEM),
            pl.BlockSpec((BB, N, F), lambda i: (i, 0, 0)),
            pl.BlockSpec((BB, P), lambda i: (i, 0)),
            full((N, N)),
            full((N, N)),
            full((N, N)),
            full((R, 2 * N)),
            full((F, W)), full((1, W)),
            full((W, W)), full((1, W)),
            full((W, E)), full((1, E)),
        ],
        out_specs=(
            pl.BlockSpec((BB, P), lambda i: (i, 0)),
            pl.BlockSpec((BB, R, _LANE), lambda i: (i, 0, 0)),
        ),
        compiler_params=pltpu.CompilerParams(
            dimension_semantics=("parallel",),
            vmem_limit_bytes=64 * 1024 * 1024,
        ),
        cost_estimate=cost,
    )(thr, nf, scf, idx, m1, m2, s12, w0, b0r, w1, b1r, w2, b2r)

    sc_out = sc_out.reshape(B1, B2, P)
    dists = dist.reshape(B, R * _LANE)[:, :P].reshape(B1, B2, P)
    return sc_out, dists


# speedup vs baseline: 2.7041x; 1.0058x over previous
import functools

import numpy as np
import jax
import jax.numpy as jnp
from jax.experimental import pallas as pl
from jax.experimental.pallas import tpu as pltpu

_LANE = 128
_BB = 64


def _round_up(x, m):
    return int(pl.cdiv(x, m) * m)


@functools.lru_cache(maxsize=None)
def _triu_tables(N, P_pad):
    assert N == _LANE
    R = P_pad // _LANE
    idx = np.zeros((N, N), np.int32)
    m1 = np.zeros((N, N), np.float32)
    m2 = np.zeros((N, N), np.float32)
    s1 = np.zeros((R, N), np.float32)
    s2 = np.zeros((R, N), np.float32)
    start = 0
    for i in range(N - 1):
        ln = N - 1 - i
        l0 = start % _LANE
        r0 = start // _LANE
        shift = (l0 - (i + 1)) % N
        idx[i, :] = (np.arange(N) - shift) % N
        len1 = min(ln, _LANE - l0)
        m1[i, l0:l0 + len1] = 1.0
        s1[r0, i] = 1.0
        if ln > len1:
            m2[i, 0:ln - len1] = 1.0
            s2[r0 + 1, i] = 1.0
        start += ln
    s12 = np.concatenate([s1, s2], axis=1)
    return idx, m1, m2, s12


def _make_body(BB, N, F, E):
    def _body(thr_ref, nf_ref, sc_ref, idx_ref, m1_ref, m2_ref, s12_ref,
              w0_ref, b0_ref, w1_ref, b1_ref, w2_ref, b2_ref,
              sc_out_ref, dist_ref):
        thr = thr_ref[0, 0]
        sc_out_ref[...] = sc_ref[...] - thr

        bf16 = jnp.bfloat16
        x = nf_ref[...].reshape(BB * N, F).astype(bf16)
        h = jnp.dot(x, w0_ref[...].astype(bf16),
                    preferred_element_type=jnp.float32)
        h = jnp.maximum(h.astype(bf16) + b0_ref[...].astype(bf16), 0.0)
        h = jnp.dot(h, w1_ref[...].astype(bf16),
                    preferred_element_type=jnp.float32)
        h = jnp.maximum(h.astype(bf16) + b1_ref[...].astype(bf16), 0.0)
        h = jnp.dot(h, w2_ref[...].astype(bf16),
                    preferred_element_type=jnp.float32)
        h3b = (h.astype(bf16) + b2_ref[...].astype(bf16)).reshape(BB, N, E)

        idx = idx_ref[...]
        m1 = m1_ref[...]
        m2 = m2_ref[...]

        s12 = s12_ref[...]
        jones = jnp.ones((N, E), bf16)
        cdims = (((1,), (1,)), ((), ()))
        CH = 64
        for c in range(0, BB, CH):
            parts1 = []
            parts2 = []
            for e in range(c, c + CH):
                Xb = h3b[e]
                HHb = Xb * Xb
                lhs = jnp.concatenate([HHb, jones, Xb], axis=1)
                rhs = jnp.concatenate([jones, HHb, -2.0 * Xb], axis=1)
                D = jax.lax.dot_general(
                    lhs, rhs, cdims,
                    preferred_element_type=jnp.float32)
                A = jnp.take_along_axis(D, idx, axis=1)
                Ab = A.astype(bf16)
                parts1.append(Ab * m1)
                parts2.append(Ab * m2)
            A12 = jnp.concatenate(
                [jnp.concatenate(parts1, axis=1),
                 jnp.concatenate(parts2, axis=1)], axis=0)
            out_wide = jnp.dot(s12, A12,
                               preferred_element_type=jnp.float32)
            for k in range(CH):
                dist_ref[c + k] = out_wide[:, k * _LANE:(k + 1) * _LANE]

    return _body


def kernel(sc, node_features, sc_threshold, w0, b0, w1, b1, w2, b2,
           triu_rows, triu_cols, pair_diff_t):
    B1, B2, N, F = node_features.shape
    B = B1 * B2
    P = sc.shape[-1]
    E = w2.shape[1]
    W = w0.shape[1]
    P_pad = _round_up(P, _LANE)
    R = P_pad // _LANE
    BB = _BB
    assert B % BB == 0 and N == _LANE

    idx_np, m1_np, m2_np, s12_np = _triu_tables(N, P_pad)
    idx = jnp.asarray(idx_np)
    m1 = jnp.asarray(m1_np, jnp.bfloat16)
    m2 = jnp.asarray(m2_np, jnp.bfloat16)
    s12 = jnp.asarray(s12_np, jnp.bfloat16)

    nf = node_features.reshape(B, N, F)
    scf = sc.reshape(B, P)
    thr = sc_threshold.reshape(1, 1)
    b0r = b0.reshape(1, -1)
    b1r = b1.reshape(1, -1)
    b2r = b2.reshape(1, -1)

    mlp_flops = 2 * BB * N * (F * W + W * W + W * E)
    gram_flops = BB * 2 * N * N * E
    route_flops = 2 * R * 2 * N * BB * N
    cost = pl.CostEstimate(
        flops=int((B // BB) * (mlp_flops + gram_flops + route_flops)),
        transcendentals=0,
        bytes_accessed=int(4 * (nf.size + 2 * scf.size + B * R * _LANE)),
    )

    full = lambda shape: pl.BlockSpec(shape, lambda i: tuple(0 for _ in shape))
    sc_out, dist = pl.pallas_call(
        _make_body(BB, N, F, E),
        out_shape=(jax.ShapeDtypeStruct((B, P), sc.dtype),
                   jax.ShapeDtypeStruct((B, R, _LANE), node_features.dtype)),
        grid=(B // BB,),
        in_specs=[
            pl.BlockSpec((1, 1), lambda i: (0, 0),
                         memory_space=pltpu.MemorySpace.SMEM),
            pl.BlockSpec((BB, N, F), lambda i: (i, 0, 0)),
            pl.BlockSpec((BB, P), lambda i: (i, 0)),
            full((N, N)),
            full((N, N)),
            full((N, N)),
            full((R, 2 * N)),
            full((F, W)), full((1, W)),
            full((W, W)), full((1, W)),
            full((W, E)), full((1, E)),
        ],
        out_specs=(
            pl.BlockSpec((BB, P), lambda i: (i, 0)),
            pl.BlockSpec((BB, R, _LANE), lambda i: (i, 0, 0)),
        ),
        compiler_params=pltpu.CompilerParams(
            dimension_semantics=("parallel",),
            vmem_limit_bytes=64 * 1024 * 1024,
        ),
        cost_estimate=cost,
    )(thr, nf, scf, idx, m1, m2, s12, w0, b0r, w1, b1r, w2, b2r)

    sc_out = sc_out.reshape(B1, B2, P)
    dists = dist.reshape(B, R * _LANE)[:, :P].reshape(B1, B2, P)
    return sc_out, dists
